# Initial kernel scaffold; baseline (speedup 1.0000x reference)
import jax
import jax.numpy as jnp
from jax.experimental import pallas as pl


def kernel(output_embeddings, target_ids, all_item_embeddings, supervision_weights):
    raise NotImplementedError("write your pallas kernel here")


# speedup vs baseline: 5.5653x; 5.5653x over previous
import functools

import jax
import jax.numpy as jnp
from jax import lax
from jax.experimental import pallas as pl
from jax.experimental.pallas import tpu as pltpu
from jax.experimental.pallas import tpu_sc as plsc

NUM_NEGATIVES = 100
TEMPERATURE = 0.05

_D = 64
_K = 104
_KG = 7


def _normalize_rows_body(x_ref, o_ref):
    x = x_ref[...]
    n = jnp.sqrt(jnp.sum(x * x, axis=1, keepdims=True))
    o_ref[...] = x / jnp.maximum(n, 1e-12)


def _normalize_rows(x, block_rows):
    rows, d = x.shape
    grid = (rows + block_rows - 1) // block_rows
    return pl.pallas_call(
        _normalize_rows_body,
        grid=(grid,),
        in_specs=[pl.BlockSpec((block_rows, d), lambda i: (i, 0))],
        out_specs=pl.BlockSpec((block_rows, d), lambda i: (i, 0)),
        out_shape=jax.ShapeDtypeStruct((rows, d), x.dtype),
    )(x)


@functools.lru_cache(maxsize=None)
def _make_sc_call(n_tok):
    mesh = plsc.VectorSubcoreMesh(core_axis_name="c", subcore_axis_name="s")
    nc, ns = mesh.num_cores, mesh.num_subcores
    nw = nc * ns
    ntok_w = n_tok // nw
    npairs = ntok_w // 2

    def body(table, idx, vtab, out, idx_v, v_v, rows_v, out_v, sem0, sem1):
        wid = lax.axis_index("s") * nc + lax.axis_index("c")
        base = pl.multiple_of(wid * ntok_w, 8)

        zero = jnp.zeros((16,), jnp.float32)
        for sl in range(2):
            for r in range(_K, 16 * _KG):
                for c in range(_D // 16):
                    rows_v[sl, r, pl.ds(c * 16, 16)] = zero

        kvecs = [lax.iota(jnp.int32, 16) + 16 * g for g in range(_KG)]
        mask_last = lax.iota(jnp.int32, 16) < (NUM_NEGATIVES + 1 - 16 * (_KG - 1))

        pltpu.sync_copy(idx.at[pl.ds(base, ntok_w)], idx_v)
        pltpu.sync_copy(vtab.at[pl.ds(base, ntok_w)], v_v)

        sems = (sem0, sem1)

        def issue(j, slot):
            return pltpu.async_copy(
                table.at[idx_v.at[j]], rows_v.at[slot, pl.ds(0, _K)], sems[slot]
            )

        def wait(j, slot):
            pltpu.make_async_copy(
                table.at[idx_v.at[j]], rows_v.at[slot, pl.ds(0, _K)], sems[slot]
            ).wait()

        def compute(j, slot):
            slotv = jnp.full((16,), slot, jnp.int32)

            def cbody(c, accs):
                vv = v_v[j, pl.ds(c * 16, 16)]
                accs = list(accs)
                for l in range(16):
                    vd = vv[l]
                    dv = jnp.full((16,), c * 16 + l, jnp.int32)
                    for g in range(_KG):
                        accs[g] = accs[g] + vd * plsc.load_gather(
                            rows_v, [slotv, kvecs[g], dv]
                        )
                return tuple(accs)

            accs = lax.fori_loop(
                0, _D // 16, cbody,
                tuple(jnp.zeros((16,), jnp.float32) for _ in range(_KG)),
            )
            logits0 = accs[0] * (1.0 / TEMPERATURE)
            terms = [jnp.exp(a * (1.0 / TEMPERATURE)) for a in accs]
            t_last = jnp.where(mask_last, terms[_KG - 1], 0.0)
            ssum = t_last
            for g in range(_KG - 1):
                ssum = ssum + terms[g]
            r = lax.rem(j, 8)
            out_v[r, pl.ds(0, 16)] = ssum
            out_v[r, pl.ds(16, 16)] = logits0

        issue(0, 0)

        def pair_body(jj, carry):
            j0 = 2 * jj
            j1 = j0 + 1
            j2 = j0 + 2
            wait(j0, 0)
            issue(j1, 1)
            compute(j0, 0)
            wait(j1, 1)

            @pl.when(j2 < ntok_w)
            def _():
                issue(j2, 0)

            compute(j1, 1)

            @pl.when(lax.rem(jj, 4) == 3)
            def _():
                pltpu.sync_copy(
                    out_v, out.at[pl.ds(pl.multiple_of(base + j0 - 6, 8), 8)]
                )

            return carry

        lax.fori_loop(0, npairs, pair_body, 0)

    return pl.kernel(
        body,
        out_type=jax.ShapeDtypeStruct((n_tok, 32), jnp.float32),
        mesh=mesh,
        compiler_params=pltpu.CompilerParams(
            needs_layout_passes=False, use_tc_tiling_on_sc=False
        ),
        scratch_types=[
            pltpu.VMEM((ntok_w, _K), jnp.int32),
            pltpu.VMEM((ntok_w, _D), jnp.float32),
            pltpu.VMEM((2, 16 * _KG, _D), jnp.float32),
            pltpu.VMEM((8, 32), jnp.float32),
            pltpu.SemaphoreType.DMA,
            pltpu.SemaphoreType.DMA,
        ],
    )


def _final_body(s_ref, w_ref, o_ref):
    s = s_ref[...]
    w = w_ref[...]
    ssum = jnp.sum(s[:, 0:16], axis=1, keepdims=True)
    loss = jnp.log(ssum) - s[:, 16:17]
    wcol = w[:, 0:1]
    num = jnp.sum(loss * wcol)
    den = jnp.sum(wcol)
    o_ref[...] = jnp.reshape(num / den, (1, 1))


def _final_call(sc_out, w32):
    return pl.pallas_call(
        _final_body,
        out_shape=jax.ShapeDtypeStruct((1, 1), jnp.float32),
    )(sc_out, w32)


def kernel(output_embeddings, target_ids, all_item_embeddings, supervision_weights):
    b, s, d = output_embeddings.shape
    n = b * s
    num_items = all_item_embeddings.shape[0] - 1

    flat_output = output_embeddings.reshape(-1, d)
    flat_targets = target_ids.reshape(-1)
    flat_weights = supervision_weights.reshape(-1)

    nk = jax.random.key(12345)
    nk1, nk2 = jax.random.split(nk)
    neg = jax.random.randint(nk1, (n, NUM_NEGATIVES), 1, num_items + 1)
    res = jax.random.randint(nk2, (n, NUM_NEGATIVES), 1, num_items + 1)
    neg = jnp.where(neg != flat_targets[:, None], neg, res)
    neg_idx = jnp.clip(neg - 1, 0, num_items)
    tgt_idx = jnp.clip(flat_targets - 1, 0, num_items)
    pad = jnp.zeros((n, _K - 1 - NUM_NEGATIVES), jnp.int32)
    idx_all = jnp.concatenate(
        [tgt_idx[:, None], neg_idx, pad], axis=1
    ).astype(jnp.int32)

    norm_table = _normalize_rows(all_item_embeddings, 1024)
    vnorm = _normalize_rows(flat_output, 2048)

    sc_out = _make_sc_call(n)(norm_table, idx_all, vnorm)

    w32 = jnp.broadcast_to(flat_weights[:, None], (n, 32))
    return _final_call(sc_out, w32)[0, 0]
